# Initial kernel scaffold; baseline (speedup 1.0000x reference)
import jax
import jax.numpy as jnp
from jax.experimental import pallas as pl


def kernel(feature, edge_index, edge_weight, W):
    raise NotImplementedError("write your pallas kernel here")


# speedup vs baseline: 3.3333x; 3.3333x over previous
import functools

import jax
import jax.numpy as jnp
from jax import lax
from jax.experimental import pallas as pl
from jax.experimental.pallas import tpu as pltpu
from jax.experimental.pallas import tpu_sc as plsc

N = 10000
E = 320000
D = 128

NS = 16
EPT = E // NS
C = 80
NCH = EPT // C
G = 25
NG = NCH // G
CPR = 80
NCHK = N // CPR
TPC = -(-NCHK // NS)


def _gcn_sc(x, srcr, dstr, wr):
    @functools.partial(
        pl.kernel,
        out_type=(
            jax.ShapeDtypeStruct((N, D), jnp.float32),
            jax.ShapeDtypeStruct((N, D), jnp.float32),
        ),
        mesh=plsc.VectorSubcoreMesh(
            core_axis_name="c", subcore_axis_name="s", num_cores=1),
        scratch_types=[
            pltpu.VMEM((G, C), jnp.int32),
            pltpu.VMEM((G, C), jnp.int32),
            pltpu.VMEM((G, C), jnp.float32),
            pltpu.VMEM((C, D), jnp.float32),
            pltpu.VMEM((CPR, D), jnp.float32),
            pltpu.VMEM_SHARED((N, D), jnp.float32),
            pltpu.SemaphoreType.DMA,
        ],
    )
    def gcn(x_hbm, src_hbm, dst_hbm, w_hbm, out1_hbm, out2_hbm,
            src_v, dst_v, w_v, rows_v, cbuf_v, acc_sh, sem):
        s = lax.axis_index("s")

        zero = jnp.zeros((16,), jnp.float32)

        def one_pass(p, carry):
            def zrow(r, c2):
                for b in range(D // 16):
                    cbuf_v[r, pl.ds(b * 16, 16)] = zero
                return c2

            lax.fori_loop(0, CPR, zrow, 0)

            def zchunk(t, c2):
                m = t * NS + s
                @pl.when(m < NCHK)
                def _():
                    pltpu.sync_copy(cbuf_v, acc_sh.at[pl.ds(m * CPR, CPR)])
                return c2

            lax.fori_loop(0, TPC, zchunk, 0)
            plsc.subcore_barrier()

            def group_body(gi, c2):
                pltpu.sync_copy(src_hbm.at[s, gi], src_v)
                pltpu.sync_copy(dst_hbm.at[s, gi], dst_v)
                pltpu.sync_copy(w_hbm.at[s, gi], w_v)

                def chunk_body(j, c3):
                    @pl.when(p == 0)
                    def _():
                        pltpu.async_copy(
                            x_hbm.at[src_v.at[j]], rows_v, sem).wait()

                    @pl.when(p != 0)
                    def _():
                        pltpu.async_copy(
                            out1_hbm.at[src_v.at[j]], rows_v, sem).wait()

                    def scale(g, c4):
                        wvec = w_v[j, pl.ds(g * 16, 16)]
                        for l in range(16):
                            i = g * 16 + l
                            wi = wvec[l]
                            for b in range(D // 16):
                                rows_v[i, pl.ds(b * 16, 16)] = (
                                    rows_v[i, pl.ds(b * 16, 16)] * wi)
                        return c4

                    lax.fori_loop(0, C // 16, scale, 0)
                    pltpu.sync_copy(rows_v, acc_sh.at[dst_v.at[j]], add=True)
                    return c3

                lax.fori_loop(0, G, chunk_body, 0)
                return c2

            lax.fori_loop(0, NG, group_body, 0)
            plsc.subcore_barrier()

            def cchunk(t, c2):
                m = t * NS + s
                @pl.when(m < NCHK)
                def _():
                    pltpu.sync_copy(acc_sh.at[pl.ds(m * CPR, CPR)], cbuf_v)
                    @pl.when(p == 0)
                    def _():
                        pltpu.sync_copy(
                            cbuf_v, out1_hbm.at[pl.ds(m * CPR, CPR)])
                    @pl.when(p != 0)
                    def _():
                        pltpu.sync_copy(
                            cbuf_v, out2_hbm.at[pl.ds(m * CPR, CPR)])
                return c2

            lax.fori_loop(0, TPC, cchunk, 0)
            plsc.subcore_barrier()
            return carry

        lax.fori_loop(0, 2, one_pass, 0)

    return gcn(x, srcr, dstr, wr)


_BR = 1000


def _final_tc(q, f, w):
    def body(q_ref, f_ref, w_ref, o_ref):
        x = 0.5 * q_ref[...] - 0.5 * f_ref[...]
        o_ref[...] = jnp.dot(x, w_ref[...], preferred_element_type=jnp.float32)

    return pl.pallas_call(
        body,
        grid=(N // _BR,),
        in_specs=[
            pl.BlockSpec((_BR, D), lambda i: (i, 0)),
            pl.BlockSpec((_BR, D), lambda i: (i, 0)),
            pl.BlockSpec((D, D), lambda i: (0, 0)),
        ],
        out_specs=pl.BlockSpec((_BR, D), lambda i: (i, 0)),
        out_shape=jax.ShapeDtypeStruct((N, D), jnp.float32),
    )(q, f, w)


def kernel(feature, edge_index, edge_weight, W):
    src = edge_index[1].astype(jnp.int32).reshape(NS, NG, G, C)
    dst = edge_index[0].astype(jnp.int32).reshape(NS, NG, G, C)
    wr = edge_weight.astype(jnp.float32).reshape(NS, NG, G, C)

    _, out2 = _gcn_sc(feature, src, dst, wr)
    return _final_tc(out2, feature, W)


# speedup vs baseline: 4.5373x; 1.3612x over previous
import functools

import jax
import jax.numpy as jnp
from jax import lax
from jax.experimental import pallas as pl
from jax.experimental.pallas import tpu as pltpu
from jax.experimental.pallas import tpu_sc as plsc

N = 10000
E = 320000
D = 128

NS = 16
EPT = E // NS
C = 64
NCH = 315
EPTP = NCH * C
G = 21
NG = NCH // G
NBUF = 3
CPR = 80
NCHK = N // CPR
TPC = -(-NCHK // NS)


def _gcn_sc(x, srcr, dstr, wr):
    @functools.partial(
        pl.kernel,
        out_type=(
            jax.ShapeDtypeStruct((N, D), jnp.float32),
            jax.ShapeDtypeStruct((N, D), jnp.float32),
        ),
        mesh=plsc.VectorSubcoreMesh(
            core_axis_name="c", subcore_axis_name="s", num_cores=1),
        scratch_types=[
            pltpu.VMEM((G, C), jnp.int32),
            pltpu.VMEM((G, C), jnp.int32),
            pltpu.VMEM((G, C), jnp.float32),
            pltpu.VMEM((NBUF, C, D), jnp.float32),
            pltpu.VMEM((CPR, D), jnp.float32),
            pltpu.VMEM_SHARED((N, D), jnp.float32),
            pltpu.SemaphoreType.DMA,
            pltpu.SemaphoreType.DMA,
        ],
    )
    def gcn(x_hbm, src_hbm, dst_hbm, w_hbm, out1_hbm, out2_hbm,
            src_v, dst_v, w_v, rows_v, cbuf_v, acc_sh, gsem, ssem):
        s = lax.axis_index("s")

        zero = jnp.zeros((16,), jnp.float32)

        def issue_gather(p, row, buf):
            @pl.when(p == 0)
            def _():
                pltpu.async_copy(
                    x_hbm.at[src_v.at[row]], rows_v.at[buf], gsem)

            @pl.when(p != 0)
            def _():
                pltpu.async_copy(
                    out1_hbm.at[src_v.at[row]], rows_v.at[buf], gsem)

        def wait_gather(buf):
            pltpu.make_async_copy(
                x_hbm.at[pl.ds(0, C)], rows_v.at[buf], gsem).wait()

        def issue_scatter(row, buf):
            pltpu.async_copy(
                rows_v.at[buf], acc_sh.at[dst_v.at[row]], ssem, add=True)

        def wait_scatter():
            pltpu.make_async_copy(
                rows_v.at[0], acc_sh.at[pl.ds(0, C)], ssem).wait()

        def scale(row, buf):
            def sgroup(g, c4):
                wvec = w_v[row, pl.ds(g * 16, 16)]
                for l in range(16):
                    i = g * 16 + l
                    wi = wvec[l]
                    for b in range(D // 16):
                        rows_v[buf, i, pl.ds(b * 16, 16)] = (
                            rows_v[buf, i, pl.ds(b * 16, 16)] * wi)
                return c4

            lax.fori_loop(0, C // 16, sgroup, 0)

        def one_pass(p, carry):
            def zrow(r, c2):
                for b in range(D // 16):
                    cbuf_v[r, pl.ds(b * 16, 16)] = zero
                return c2

            lax.fori_loop(0, CPR, zrow, 0)

            def zchunk(t, c2):
                m = t * NS + s
                @pl.when(m < NCHK)
                def _():
                    pltpu.sync_copy(cbuf_v, acc_sh.at[pl.ds(m * CPR, CPR)])
                return c2

            lax.fori_loop(0, TPC, zchunk, 0)
            plsc.subcore_barrier()

            def group_body(gi, c2):
                pltpu.sync_copy(src_hbm.at[s, gi], src_v)
                pltpu.sync_copy(dst_hbm.at[s, gi], dst_v)
                pltpu.sync_copy(w_hbm.at[s, gi], w_v)

                issue_gather(p, 0, 0)
                issue_gather(p, 1, 1)

                def triple(t, c3):
                    for b in range(NBUF):
                        j = t * NBUF + b
                        if b == 0:
                            @pl.when(t > 0)
                            def _():
                                wait_scatter()
                        else:
                            wait_scatter()
                        if b == 0:
                            issue_gather(p, j + 2, (b + 2) % NBUF)
                        else:
                            @pl.when(t < G // NBUF - 1)
                            def _():
                                issue_gather(p, j + 2, (b + 2) % NBUF)
                        wait_gather(b)
                        scale(j, b)
                        issue_scatter(j, b)
                    return c3

                lax.fori_loop(0, G // NBUF, triple, 0)
                wait_scatter()
                return c2

            lax.fori_loop(0, NG, group_body, 0)
            plsc.subcore_barrier()

            def cchunk(t, c2):
                m = t * NS + s
                @pl.when(m < NCHK)
                def _():
                    pltpu.sync_copy(acc_sh.at[pl.ds(m * CPR, CPR)], cbuf_v)
                    @pl.when(p == 0)
                    def _():
                        pltpu.sync_copy(
                            cbuf_v, out1_hbm.at[pl.ds(m * CPR, CPR)])
                    @pl.when(p != 0)
                    def _():
                        pltpu.sync_copy(
                            cbuf_v, out2_hbm.at[pl.ds(m * CPR, CPR)])
                return c2

            lax.fori_loop(0, TPC, cchunk, 0)
            plsc.subcore_barrier()
            return carry

        lax.fori_loop(0, 2, one_pass, 0)

    return gcn(x, srcr, dstr, wr)


_BR = 1000


def _final_tc(q, f, w):
    def body(q_ref, f_ref, w_ref, o_ref):
        x = 0.5 * q_ref[...] - 0.5 * f_ref[...]
        o_ref[...] = jnp.dot(x, w_ref[...], preferred_element_type=jnp.float32)

    return pl.pallas_call(
        body,
        grid=(N // _BR,),
        in_specs=[
            pl.BlockSpec((_BR, D), lambda i: (i, 0)),
            pl.BlockSpec((_BR, D), lambda i: (i, 0)),
            pl.BlockSpec((D, D), lambda i: (0, 0)),
        ],
        out_specs=pl.BlockSpec((_BR, D), lambda i: (i, 0)),
        out_shape=jax.ShapeDtypeStruct((N, D), jnp.float32),
    )(q, f, w)


def kernel(feature, edge_index, edge_weight, W):
    pad = ((0, 0), (0, EPTP - EPT))
    src = jnp.pad(edge_index[1].astype(jnp.int32).reshape(NS, EPT), pad)
    dst = jnp.pad(edge_index[0].astype(jnp.int32).reshape(NS, EPT), pad)
    wr = jnp.pad(edge_weight.astype(jnp.float32).reshape(NS, EPT), pad)
    src = src.reshape(NS, NG, G, C)
    dst = dst.reshape(NS, NG, G, C)
    wr = wr.reshape(NS, NG, G, C)

    _, out2 = _gcn_sc(feature, src, dst, wr)
    return _final_tc(out2, feature, W)


# speedup vs baseline: 6.1985x; 1.3661x over previous
import functools

import jax
import jax.numpy as jnp
from jax import lax
from jax.experimental import pallas as pl
from jax.experimental.pallas import tpu as pltpu
from jax.experimental.pallas import tpu_sc as plsc

N = 10000
E = 320000
D = 128

NC = 2
NS = 16
NW = NC * NS
EPT = E // NW
C = 80
NCH = 126
EPTP = NCH * C
G = 9
NG = NCH // G
NBUF = 3
CPR = 80
NCHK = N // CPR
TPC = -(-NCHK // NS)


def _spmm_sc(x, srcr, dstr, wr):
    @functools.partial(
        pl.kernel,
        out_type=jax.ShapeDtypeStruct((NC, N, D), jnp.float32),
        mesh=plsc.VectorSubcoreMesh(
            core_axis_name="c", subcore_axis_name="s", num_cores=NC),
        scratch_types=[
            pltpu.VMEM((G, C), jnp.int32),
            pltpu.VMEM((G, C), jnp.int32),
            pltpu.VMEM((G, C), jnp.float32),
            pltpu.VMEM((NBUF, C, D), jnp.float32),
            pltpu.VMEM((CPR, D), jnp.float32),
            pltpu.VMEM_SHARED((N, D), jnp.float32),
            pltpu.SemaphoreType.DMA,
            pltpu.SemaphoreType.DMA,
        ],
    )
    def spmm(x_hbm, src_hbm, dst_hbm, w_hbm, out_hbm,
             src_v, dst_v, w_v, rows_v, cbuf_v, acc_sh, gsem, ssem):
        c = lax.axis_index("c")
        s = lax.axis_index("s")
        wid = c * NS + s

        zero = jnp.zeros((16,), jnp.float32)

        def issue_gather(row, buf):
            pltpu.async_copy(x_hbm.at[src_v.at[row]], rows_v.at[buf], gsem)

        def wait_gather(buf):
            pltpu.make_async_copy(
                x_hbm.at[pl.ds(0, C)], rows_v.at[buf], gsem).wait()

        def issue_scatter(row, buf):
            pltpu.async_copy(
                rows_v.at[buf], acc_sh.at[dst_v.at[row]], ssem, add=True)

        def wait_scatter():
            pltpu.make_async_copy(
                rows_v.at[0], acc_sh.at[pl.ds(0, C)], ssem).wait()

        def scale(row, buf):
            def sgroup(g, c4):
                wvec = w_v[row, pl.ds(g * 16, 16)]
                for l in range(16):
                    i = g * 16 + l
                    wi = wvec[l]
                    for b in range(D // 16):
                        rows_v[buf, i, pl.ds(b * 16, 16)] = (
                            rows_v[buf, i, pl.ds(b * 16, 16)] * wi)
                return c4

            lax.fori_loop(0, C // 16, sgroup, 0)

        def zrow(r, c2):
            for b in range(D // 16):
                cbuf_v[r, pl.ds(b * 16, 16)] = zero
            return c2

        lax.fori_loop(0, CPR, zrow, 0)

        def zchunk(t, c2):
            m = t * NS + s
            @pl.when(m < NCHK)
            def _():
                pltpu.sync_copy(cbuf_v, acc_sh.at[pl.ds(m * CPR, CPR)])
            return c2

        lax.fori_loop(0, TPC, zchunk, 0)
        plsc.subcore_barrier()

        def group_body(gi, c2):
            pltpu.sync_copy(src_hbm.at[wid, gi], src_v)
            pltpu.sync_copy(dst_hbm.at[wid, gi], dst_v)
            pltpu.sync_copy(w_hbm.at[wid, gi], w_v)

            issue_gather(0, 0)
            issue_gather(1, 1)

            def triple(t, c3):
                for b in range(NBUF):
                    j = t * NBUF + b
                    if b == 0:
                        @pl.when(t > 0)
                        def _():
                            wait_scatter()
                    else:
                        wait_scatter()
                    if b == 0:
                        issue_gather(j + 2, (b + 2) % NBUF)
                    else:
                        @pl.when(t < G // NBUF - 1)
                        def _():
                            issue_gather(j + 2, (b + 2) % NBUF)
                    wait_gather(b)
                    scale(j, b)
                    issue_scatter(j, b)
                return c3

            lax.fori_loop(0, G // NBUF, triple, 0)
            wait_scatter()
            return c2

        lax.fori_loop(0, NG, group_body, 0)
        plsc.subcore_barrier()

        def cchunk(t, c2):
            m = t * NS + s
            @pl.when(m < NCHK)
            def _():
                pltpu.sync_copy(acc_sh.at[pl.ds(m * CPR, CPR)], cbuf_v)
                pltpu.sync_copy(cbuf_v, out_hbm.at[c, pl.ds(m * CPR, CPR)])
            return c2

        lax.fori_loop(0, TPC, cchunk, 0)

    return spmm(x, srcr, dstr, wr)


_BR = 1000


def _combine_tc(p):
    def body(p_ref, o_ref):
        o_ref[...] = p_ref[0] + p_ref[1]

    return pl.pallas_call(
        body,
        grid=(N // _BR,),
        in_specs=[pl.BlockSpec((NC, _BR, D), lambda i: (0, i, 0))],
        out_specs=pl.BlockSpec((_BR, D), lambda i: (i, 0)),
        out_shape=jax.ShapeDtypeStruct((N, D), jnp.float32),
    )(p)


def _final_tc(q, f, w):
    def body(q_ref, f_ref, w_ref, o_ref):
        x = 0.5 * (q_ref[0] + q_ref[1]) - 0.5 * f_ref[...]
        o_ref[...] = jnp.dot(x, w_ref[...], preferred_element_type=jnp.float32)

    return pl.pallas_call(
        body,
        grid=(N // _BR,),
        in_specs=[
            pl.BlockSpec((NC, _BR, D), lambda i: (0, i, 0)),
            pl.BlockSpec((_BR, D), lambda i: (i, 0)),
            pl.BlockSpec((D, D), lambda i: (0, 0)),
        ],
        out_specs=pl.BlockSpec((_BR, D), lambda i: (i, 0)),
        out_shape=jax.ShapeDtypeStruct((N, D), jnp.float32),
    )(q, f, w)


def kernel(feature, edge_index, edge_weight, W):
    pad = ((0, 0), (0, EPTP - EPT))
    src = jnp.pad(edge_index[1].astype(jnp.int32).reshape(NW, EPT), pad)
    dst = jnp.pad(edge_index[0].astype(jnp.int32).reshape(NW, EPT), pad)
    wr = jnp.pad(edge_weight.astype(jnp.float32).reshape(NW, EPT), pad)
    src = src.reshape(NW, NG, G, C)
    dst = dst.reshape(NW, NG, G, C)
    wr = wr.reshape(NW, NG, G, C)

    p = _spmm_sc(feature, src, dst, wr)
    out1 = _combine_tc(p)
    q = _spmm_sc(out1, src, dst, wr)
    return _final_tc(q, feature, W)


# speedup vs baseline: 6.5698x; 1.0599x over previous
import functools

import jax
import jax.numpy as jnp
from jax import lax
from jax.experimental import pallas as pl
from jax.experimental.pallas import tpu as pltpu
from jax.experimental.pallas import tpu_sc as plsc

N = 10000
E = 320000
D = 128

NC = 2
NS = 16
NW = NC * NS
EPT = E // NW
C = 80
NCH = 126
EPTP = NCH * C
G = 9
NG = NCH // G
NBUF = 3
CPR = 40
NCHK = N // CPR
TPC = -(-NCHK // NS)


def _spmm_sc(x, srcr, dstr, wr):
    @functools.partial(
        pl.kernel,
        out_type=jax.ShapeDtypeStruct((NC, N, D), jnp.float32),
        mesh=plsc.VectorSubcoreMesh(
            core_axis_name="c", subcore_axis_name="s", num_cores=NC),
        scratch_types=[
            pltpu.VMEM((2, G, C), jnp.int32),
            pltpu.VMEM((2, G, C), jnp.int32),
            pltpu.VMEM((2, G, C), jnp.float32),
            pltpu.VMEM((NBUF, C, D), jnp.float32),
            pltpu.VMEM((CPR, D), jnp.float32),
            pltpu.VMEM_SHARED((N, D), jnp.float32),
            pltpu.SemaphoreType.DMA,
            pltpu.SemaphoreType.DMA,
            pltpu.SemaphoreType.DMA,
        ],
    )
    def spmm(x_hbm, src_hbm, dst_hbm, w_hbm, out_hbm,
             src_v, dst_v, w_v, rows_v, cbuf_v, acc_sh, gsem, ssem, isem):
        c = lax.axis_index("c")
        s = lax.axis_index("s")
        wid = c * NS + s

        zero = jnp.zeros((16,), jnp.float32)

        def issue_stage(gi, slot):
            pltpu.async_copy(src_hbm.at[wid, gi], src_v.at[slot], isem)
            pltpu.async_copy(dst_hbm.at[wid, gi], dst_v.at[slot], isem)
            pltpu.async_copy(w_hbm.at[wid, gi], w_v.at[slot], isem)

        def wait_stage(slot):
            pltpu.make_async_copy(
                src_hbm.at[0, 0], src_v.at[slot], isem).wait()
            pltpu.make_async_copy(
                dst_hbm.at[0, 0], dst_v.at[slot], isem).wait()
            pltpu.make_async_copy(
                w_hbm.at[0, 0], w_v.at[slot], isem).wait()

        def issue_gather(slot, row, buf):
            pltpu.async_copy(
                x_hbm.at[src_v.at[slot, row]], rows_v.at[buf], gsem)

        def wait_gather(buf):
            pltpu.make_async_copy(
                x_hbm.at[pl.ds(0, C)], rows_v.at[buf], gsem).wait()

        def issue_scatter(slot, row, buf):
            pltpu.async_copy(
                rows_v.at[buf], acc_sh.at[dst_v.at[slot, row]], ssem,
                add=True)

        def wait_scatter():
            pltpu.make_async_copy(
                rows_v.at[0], acc_sh.at[pl.ds(0, C)], ssem).wait()

        def scale(slot, row, buf):
            def sgroup(g, c4):
                wvec = w_v[slot, row, pl.ds(g * 16, 16)]
                for l in range(16):
                    i = g * 16 + l
                    wi = wvec[l]
                    for b in range(D // 16):
                        rows_v[buf, i, pl.ds(b * 16, 16)] = (
                            rows_v[buf, i, pl.ds(b * 16, 16)] * wi)
                return c4

            lax.fori_loop(0, C // 16, sgroup, 0)

        def zrow(r, c2):
            for b in range(D // 16):
                cbuf_v[r, pl.ds(b * 16, 16)] = zero
            return c2

        lax.fori_loop(0, CPR, zrow, 0)

        def zchunk(t, c2):
            m = t * NS + s
            @pl.when(m < NCHK)
            def _():
                pltpu.sync_copy(cbuf_v, acc_sh.at[pl.ds(m * CPR, CPR)])
            return c2

        lax.fori_loop(0, TPC, zchunk, 0)
        plsc.subcore_barrier()

        def run_group(gi, slot):
            wait_stage(slot)
            issue_gather(slot, 0, 0)
            issue_gather(slot, 1, 1)

            def triple(t, c3):
                for b in range(NBUF):
                    j = t * NBUF + b
                    if b == 0:
                        @pl.when(t > 0)
                        def _():
                            wait_scatter()
                    else:
                        wait_scatter()
                    if b == 0:
                        issue_gather(slot, j + 2, (b + 2) % NBUF)
                    else:
                        @pl.when(t < G // NBUF - 1)
                        def _():
                            issue_gather(slot, j + 2, (b + 2) % NBUF)
                    wait_gather(b)
                    scale(slot, j, b)
                    issue_scatter(slot, j, b)
                return c3

            lax.fori_loop(0, G // NBUF, triple, 0)
            wait_scatter()

        issue_stage(0, 0)

        def group_pair(gp, c2):
            issue_stage(2 * gp + 1, 1)
            run_group(2 * gp, 0)
            @pl.when(gp < NG // 2 - 1)
            def _():
                issue_stage(2 * gp + 2, 0)
            run_group(2 * gp + 1, 1)
            return c2

        lax.fori_loop(0, NG // 2, group_pair, 0)
        plsc.subcore_barrier()

        def cchunk(t, c2):
            m = t * NS + s
            @pl.when(m < NCHK)
            def _():
                pltpu.sync_copy(acc_sh.at[pl.ds(m * CPR, CPR)], cbuf_v)
                pltpu.sync_copy(cbuf_v, out_hbm.at[c, pl.ds(m * CPR, CPR)])
            return c2

        lax.fori_loop(0, TPC, cchunk, 0)

    return spmm(x, srcr, dstr, wr)


_BR = 1000


def _combine_tc(p):
    def body(p_ref, o_ref):
        o_ref[...] = p_ref[0] + p_ref[1]

    return pl.pallas_call(
        body,
        grid=(N // _BR,),
        in_specs=[pl.BlockSpec((NC, _BR, D), lambda i: (0, i, 0))],
        out_specs=pl.BlockSpec((_BR, D), lambda i: (i, 0)),
        out_shape=jax.ShapeDtypeStruct((N, D), jnp.float32),
    )(p)


def _final_tc(q, f, w):
    def body(q_ref, f_ref, w_ref, o_ref):
        x = 0.5 * (q_ref[0] + q_ref[1]) - 0.5 * f_ref[...]
        o_ref[...] = jnp.dot(x, w_ref[...], preferred_element_type=jnp.float32)

    return pl.pallas_call(
        body,
        grid=(N // _BR,),
        in_specs=[
            pl.BlockSpec((NC, _BR, D), lambda i: (0, i, 0)),
            pl.BlockSpec((_BR, D), lambda i: (i, 0)),
            pl.BlockSpec((D, D), lambda i: (0, 0)),
        ],
        out_specs=pl.BlockSpec((_BR, D), lambda i: (i, 0)),
        out_shape=jax.ShapeDtypeStruct((N, D), jnp.float32),
    )(q, f, w)


def kernel(feature, edge_index, edge_weight, W):
    pad = ((0, 0), (0, EPTP - EPT))
    src = jnp.pad(edge_index[1].astype(jnp.int32).reshape(NW, EPT), pad)
    dst = jnp.pad(edge_index[0].astype(jnp.int32).reshape(NW, EPT), pad)
    wr = jnp.pad(edge_weight.astype(jnp.float32).reshape(NW, EPT), pad)
    src = src.reshape(NW, NG, G, C)
    dst = dst.reshape(NW, NG, G, C)
    wr = wr.reshape(NW, NG, G, C)

    p = _spmm_sc(feature, src, dst, wr)
    out1 = _combine_tc(p)
    q = _spmm_sc(out1, src, dst, wr)
    return _final_tc(q, feature, W)


# speedup vs baseline: 6.6978x; 1.0195x over previous
import functools

import jax
import jax.numpy as jnp
from jax import lax
from jax.experimental import pallas as pl
from jax.experimental.pallas import tpu as pltpu
from jax.experimental.pallas import tpu_sc as plsc

N = 10000
E = 320000
D = 128

NC = 2
NS = 16
NW = NC * NS
EPT = E // NW
C = 80
NCH = 126
EPTP = NCH * C
G = 9
NG = NCH // G
NBUF = 3
CPR = 40
NCHK = N // CPR
TPC = -(-NCHK // NS)


def _spmm_sc(x, srcr, dstr, wr):
    @functools.partial(
        pl.kernel,
        out_type=jax.ShapeDtypeStruct((NC, N, D), jnp.float32),
        mesh=plsc.VectorSubcoreMesh(
            core_axis_name="c", subcore_axis_name="s", num_cores=NC),
        scratch_types=[
            pltpu.VMEM((2, G, C), jnp.int32),
            pltpu.VMEM((2, G, C), jnp.int32),
            pltpu.VMEM((2, G, C), jnp.float32),
            pltpu.VMEM((NBUF, C, D), jnp.float32),
            pltpu.VMEM((CPR, D), jnp.float32),
            pltpu.VMEM_SHARED((N, D), jnp.float32),
            pltpu.SemaphoreType.DMA,
            pltpu.SemaphoreType.DMA,
            pltpu.SemaphoreType.DMA,
        ],
    )
    def spmm(x_hbm, src_hbm, dst_hbm, w_hbm, out_hbm,
             src_v, dst_v, w_v, rows_v, cbuf_v, acc_sh, gsem, ssem, isem):
        c = lax.axis_index("c")
        s = lax.axis_index("s")
        wid = c * NS + s

        zero = jnp.zeros((16,), jnp.float32)

        def issue_stage(gi, slot):
            pltpu.async_copy(src_hbm.at[wid, gi], src_v.at[slot], isem)
            pltpu.async_copy(dst_hbm.at[wid, gi], dst_v.at[slot], isem)
            pltpu.async_copy(w_hbm.at[wid, gi], w_v.at[slot], isem)

        def wait_stage(slot):
            pltpu.make_async_copy(
                src_hbm.at[0, 0], src_v.at[slot], isem).wait()
            pltpu.make_async_copy(
                dst_hbm.at[0, 0], dst_v.at[slot], isem).wait()
            pltpu.make_async_copy(
                w_hbm.at[0, 0], w_v.at[slot], isem).wait()

        def issue_gather(slot, row, buf):
            pltpu.async_copy(
                x_hbm.at[src_v.at[slot, row]], rows_v.at[buf], gsem)

        def wait_gather(buf):
            pltpu.make_async_copy(
                x_hbm.at[pl.ds(0, C)], rows_v.at[buf], gsem).wait()

        def issue_scatter(slot, row, buf):
            pltpu.async_copy(
                rows_v.at[buf], acc_sh.at[dst_v.at[slot, row]], ssem,
                add=True)

        def wait_scatter():
            pltpu.make_async_copy(
                rows_v.at[0], acc_sh.at[pl.ds(0, C)], ssem).wait()

        def scale(slot, row, buf):
            def sgroup(g, c4):
                wvec = w_v[slot, row, pl.ds(g * 16, 16)]
                for l in range(16):
                    i = g * 16 + l
                    wi = wvec[l]
                    for b in range(D // 16):
                        rows_v[buf, i, pl.ds(b * 16, 16)] = (
                            rows_v[buf, i, pl.ds(b * 16, 16)] * wi)
                return c4

            lax.fori_loop(0, C // 16, sgroup, 0)

        def zrow(r, c2):
            for b in range(D // 16):
                cbuf_v[r, pl.ds(b * 16, 16)] = zero
            return c2

        lax.fori_loop(0, CPR, zrow, 0)

        def zchunk(t, c2):
            m = t * NS + s
            @pl.when(m < NCHK)
            def _():
                pltpu.sync_copy(cbuf_v, acc_sh.at[pl.ds(m * CPR, CPR)])
            return c2

        lax.fori_loop(0, TPC, zchunk, 0)
        plsc.subcore_barrier()

        def issue_first_two(slot):
            issue_gather(slot, 0, 0)
            issue_gather(slot, 1, 1)

        def run_group(gi, slot, next_cond):

            def triple(t, c3):
                for b in range(NBUF):
                    j = t * NBUF + b
                    if b == 0:
                        @pl.when(t > 0)
                        def _():
                            wait_scatter()
                    else:
                        wait_scatter()
                    if b == 0:
                        issue_gather(slot, j + 2, (b + 2) % NBUF)
                    else:
                        @pl.when(t < G // NBUF - 1)
                        def _():
                            issue_gather(slot, j + 2, (b + 2) % NBUF)
                    wait_gather(b)
                    scale(slot, j, b)
                    issue_scatter(slot, j, b)
                return c3

            lax.fori_loop(0, G // NBUF, triple, 0)
            @pl.when(next_cond)
            def _():
                wait_stage(1 - slot)
                issue_first_two(1 - slot)
            wait_scatter()

        issue_stage(0, 0)
        wait_stage(0)
        issue_first_two(0)

        def group_pair(gp, c2):
            issue_stage(2 * gp + 1, 1)
            run_group(2 * gp, 0, jnp.bool_(True))
            @pl.when(gp < NG // 2 - 1)
            def _():
                issue_stage(2 * gp + 2, 0)
            run_group(2 * gp + 1, 1, gp < NG // 2 - 1)
            return c2

        lax.fori_loop(0, NG // 2, group_pair, 0)
        plsc.subcore_barrier()

        def cchunk(t, c2):
            m = t * NS + s
            @pl.when(m < NCHK)
            def _():
                pltpu.sync_copy(acc_sh.at[pl.ds(m * CPR, CPR)], cbuf_v)
                pltpu.sync_copy(cbuf_v, out_hbm.at[c, pl.ds(m * CPR, CPR)])
            return c2

        lax.fori_loop(0, TPC, cchunk, 0)

    return spmm(x, srcr, dstr, wr)


_BR = 1000


def _combine_tc(p):
    def body(p_ref, o_ref):
        o_ref[...] = p_ref[0] + p_ref[1]

    return pl.pallas_call(
        body,
        grid=(N // _BR,),
        in_specs=[pl.BlockSpec((NC, _BR, D), lambda i: (0, i, 0))],
        out_specs=pl.BlockSpec((_BR, D), lambda i: (i, 0)),
        out_shape=jax.ShapeDtypeStruct((N, D), jnp.float32),
    )(p)


def _final_tc(q, f, w):
    def body(q_ref, f_ref, w_ref, o_ref):
        x = 0.5 * (q_ref[0] + q_ref[1]) - 0.5 * f_ref[...]
        o_ref[...] = jnp.dot(x, w_ref[...], preferred_element_type=jnp.float32)

    return pl.pallas_call(
        body,
        grid=(N // _BR,),
        in_specs=[
            pl.BlockSpec((NC, _BR, D), lambda i: (0, i, 0)),
            pl.BlockSpec((_BR, D), lambda i: (i, 0)),
            pl.BlockSpec((D, D), lambda i: (0, 0)),
        ],
        out_specs=pl.BlockSpec((_BR, D), lambda i: (i, 0)),
        out_shape=jax.ShapeDtypeStruct((N, D), jnp.float32),
    )(q, f, w)


def kernel(feature, edge_index, edge_weight, W):
    pad = ((0, 0), (0, EPTP - EPT))
    src = jnp.pad(edge_index[1].astype(jnp.int32).reshape(NW, EPT), pad)
    dst = jnp.pad(edge_index[0].astype(jnp.int32).reshape(NW, EPT), pad)
    wr = jnp.pad(edge_weight.astype(jnp.float32).reshape(NW, EPT), pad)
    src = src.reshape(NW, NG, G, C)
    dst = dst.reshape(NW, NG, G, C)
    wr = wr.reshape(NW, NG, G, C)

    p = _spmm_sc(feature, src, dst, wr)
    out1 = _combine_tc(p)
    q = _spmm_sc(out1, src, dst, wr)
    return _final_tc(q, feature, W)
